# Initial kernel scaffold; baseline (speedup 1.0000x reference)
import functools

import jax
import jax.numpy as jnp
from jax import lax
from jax.experimental import pallas as pl
from jax.experimental.pallas import tpu as pltpu
from jax.experimental.pallas import tpu_sc as plsc

B, T, D, DFF = 4, 8192, 768, 2048
K = 1024
N = B * T
NSEL = B * K
DQ = D // 4

RB1 = 512
NBLK1 = N // RB1


def _pass1_body(h_ref, wr_ref, w1_ref, b1_ref, w2_ref, b2_ref,
                out_ref, logit_ref, pred_ref, part_ref):
    h = h_ref[...]
    out_ref[...] = h
    logits = jnp.dot(h, wr_ref[...],
                     preferred_element_type=jnp.float32)
    logit_ref[...] = logits
    pre = jnp.dot(h.astype(jnp.bfloat16), w1_ref[...],
                  preferred_element_type=jnp.float32) + b1_ref[...]
    pre = jax.nn.gelu(pre, approximate=False)
    pred = jnp.dot(pre, w2_ref[...],
                   preferred_element_type=jnp.float32) + b2_ref[...]
    pred_ref[...] = pred

    def softplus_lanes(x):
        sp = jnp.maximum(x, 0.0) + jnp.log1p(jnp.exp(-jnp.abs(x)))
        return jnp.sum(sp.reshape(RB1 // 128, 128), axis=0, keepdims=True)

    blk = jnp.concatenate([softplus_lanes(logits), softplus_lanes(pred)], 0)

    @pl.when(pl.program_id(0) == 0)
    def _():
        part_ref[...] = jnp.zeros_like(part_ref)
    part_ref[...] += blk


def _pass1(h_flat, wr2, w1b, b12, w22, b22):
    return pl.pallas_call(
        _pass1_body,
        grid=(NBLK1,),
        in_specs=[
            pl.BlockSpec((RB1, D), lambda i: (i, 0)),
            pl.BlockSpec((D, 1), lambda i: (0, 0)),
            pl.BlockSpec((D, DQ), lambda i: (0, 0)),
            pl.BlockSpec((1, DQ), lambda i: (0, 0)),
            pl.BlockSpec((DQ, 1), lambda i: (0, 0)),
            pl.BlockSpec((1, 1), lambda i: (0, 0)),
        ],
        out_specs=[
            pl.BlockSpec((RB1, D), lambda i: (i, 0)),
            pl.BlockSpec((RB1, 1), lambda i: (i, 0)),
            pl.BlockSpec((RB1, 1), lambda i: (i, 0)),
            pl.BlockSpec((2, 128), lambda i: (0, 0)),
        ],
        out_shape=[
            jax.ShapeDtypeStruct((N, D), jnp.float32),
            jax.ShapeDtypeStruct((N, 1), jnp.float32),
            jax.ShapeDtypeStruct((N, 1), jnp.float32),
            jax.ShapeDtypeStruct((2, 128), jnp.float32),
        ],
    )(h_flat, wr2, w1b, b12, w22, b22)


_MESH = plsc.VectorSubcoreMesh(core_axis_name="c", subcore_axis_name="s",
                               num_cores=2, num_subcores=16)
_INT_MIN = jnp.int32(-2**31)


def _f32_key(v):
    b = plsc.bitcast(v, jnp.int32)
    return b ^ ((b >> 31) & jnp.int32(0x7FFFFFFF))


def _select_body(logits_hbm, pred_hbm, idx_out, gate_out, psel_out,
                 lrow, prow, idxs, vals, pvals):
    wid = lax.axis_index("s") * 2 + lax.axis_index("c")

    @pl.when(wid < B)
    def _():
        row = wid
        pltpu.sync_copy(logits_hbm.at[pl.ds(row * T, T)], lrow)
        pltpu.sync_copy(pred_hbm.at[pl.ds(row * T, T)], prow)

        nv = T // 16

        def count_ge(thresh):
            tv = jnp.full((16,), thresh, jnp.int32)

            def body(i, acc):
                key = _f32_key(lrow[pl.ds(i * 16, 16)])
                return acc + jnp.where(key >= tv, 1, 0).astype(jnp.int32)

            acc = lax.fori_loop(0, nv, body,
                                jnp.zeros((16,), jnp.int32), unroll=8)
            return jnp.sum(acc)

        prefix = jnp.where(count_ge(jnp.int32(0)) >= K, jnp.int32(0), _INT_MIN)

        def bit_body(j, prefix):
            test = prefix + (jnp.int32(1) << (jnp.int32(30) - j))
            return jnp.where(count_ge(test) >= K, test, prefix)

        tau = lax.fori_loop(0, 31, bit_body, prefix)
        c1 = count_ge(tau + 1)
        need = K - c1

        tau_v = jnp.full((16,), tau, jnp.int32)
        lane = lax.broadcasted_iota(jnp.int32, (16,), 0)

        def sweep(i, carry):
            s, eqc = carry
            v = lrow[pl.ds(i * 16, 16)]
            p = prow[pl.ds(i * 16, 16)]
            key = _f32_key(v)
            gt = key > tau_v
            eq = key == tau_v
            eqi = jnp.where(eq, 1, 0).astype(jnp.int32)
            eqpref = plsc.cumsum(eqi)
            sel = gt | (eq & ((eqc + eqpref) <= need))
            seli = jnp.where(sel, 1, 0).astype(jnp.int32)
            spref = plsc.cumsum(seli)
            pos = s + spref - 1
            tok = lane + (i * 16 + row * T)
            plsc.store_scatter(vals, [pos], v, mask=sel)
            plsc.store_scatter(pvals, [pos], p, mask=sel)
            plsc.store_scatter(idxs, [pos], tok, mask=sel)
            return s + jnp.sum(seli), eqc + jnp.sum(eqi)

        lax.fori_loop(0, nv, sweep, (jnp.int32(0), jnp.int32(0)), unroll=4)

        pltpu.sync_copy(idxs, idx_out.at[pl.ds(row * K, K)])
        pltpu.sync_copy(vals, gate_out.at[pl.ds(row * K, K)])
        pltpu.sync_copy(pvals, psel_out.at[pl.ds(row * K, K)])


_select = functools.partial(
    pl.kernel,
    out_type=(
        jax.ShapeDtypeStruct((NSEL,), jnp.int32),
        jax.ShapeDtypeStruct((NSEL,), jnp.float32),
        jax.ShapeDtypeStruct((NSEL,), jnp.float32),
    ),
    mesh=_MESH,
    scratch_types=[
        pltpu.VMEM((T,), jnp.float32),
        pltpu.VMEM((T,), jnp.float32),
        pltpu.VMEM((K,), jnp.int32),
        pltpu.VMEM((K,), jnp.float32),
        pltpu.VMEM((K,), jnp.float32),
    ],
)(_select_body)


RPW = NSEL // 32


def _gather_body(hid_hbm, idx_hbm, out_hbm, idx_v, rows_v, sem):
    wid = lax.axis_index("s") * 2 + lax.axis_index("c")
    base = wid * RPW
    pltpu.sync_copy(idx_hbm.at[pl.ds(base, RPW)], idx_v)
    pltpu.async_copy(hid_hbm.at[idx_v], rows_v, sem).wait()
    pltpu.sync_copy(rows_v, out_hbm.at[pl.ds(base, RPW)])


_gather = functools.partial(
    pl.kernel,
    out_type=jax.ShapeDtypeStruct((NSEL, D), jnp.float32),
    mesh=_MESH,
    scratch_types=[
        pltpu.VMEM((RPW,), jnp.int32),
        pltpu.VMEM((RPW, D), jnp.float32),
        pltpu.SemaphoreType.DMA,
    ],
)(_gather_body)


def _scatter_body(upd_hbm, idx_hbm, states_ref, idx_v, rows_v, sem):
    wid = lax.axis_index("s") * 2 + lax.axis_index("c")
    base = wid * RPW
    pltpu.sync_copy(idx_hbm.at[pl.ds(base, RPW)], idx_v)
    pltpu.sync_copy(upd_hbm.at[pl.ds(base, RPW)], rows_v)
    pltpu.async_copy(rows_v, states_ref.at[idx_v], sem).wait()


_scatter = functools.partial(
    pl.kernel,
    out_type=(),
    mesh=_MESH,
    scratch_types=[
        pltpu.VMEM((RPW,), jnp.int32),
        pltpu.VMEM((RPW, D), jnp.float32),
        pltpu.SemaphoreType.DMA,
    ],
)(_scatter_body)


RB2 = 512
NBLK2 = NSEL // RB2


def _mlp_body(x_ref, g_ref, nw_ref, wg_ref, wu_ref, wd_ref, out_ref):
    x = x_ref[...]
    var = jnp.mean(x * x, axis=1, keepdims=True)
    h = (x * lax.rsqrt(var + 1e-6) * nw_ref[...]).astype(jnp.bfloat16)
    gproj = jnp.dot(h, wg_ref[...], preferred_element_type=jnp.float32)
    up = jnp.dot(h, wu_ref[...], preferred_element_type=jnp.float32)
    act = (gproj * jax.nn.sigmoid(gproj)) * up
    mlp = jnp.dot(act.astype(jnp.bfloat16), wd_ref[...],
                  preferred_element_type=jnp.float32)
    g = jax.nn.sigmoid(g_ref[...])
    out_ref[...] = x + g * mlp


def _mlp(sel, gate2, nw2, wgb, wub, wdb):
    return pl.pallas_call(
        _mlp_body,
        grid=(NBLK2,),
        in_specs=[
            pl.BlockSpec((RB2, D), lambda i: (i, 0)),
            pl.BlockSpec((RB2, 1), lambda i: (i, 0)),
            pl.BlockSpec((1, D), lambda i: (0, 0)),
            pl.BlockSpec((D, DFF), lambda i: (0, 0)),
            pl.BlockSpec((D, DFF), lambda i: (0, 0)),
            pl.BlockSpec((DFF, D), lambda i: (0, 0)),
        ],
        out_specs=pl.BlockSpec((RB2, D), lambda i: (i, 0)),
        out_shape=jax.ShapeDtypeStruct((NSEL, D), jnp.float32),
    )(sel, gate2, nw2, wgb, wub, wdb)


def kernel(hidden_states, Wr, W1, b1, W2, b2, norm_w, Wg, Wu, Wd, training):
    h_flat = hidden_states.reshape(N, D)
    states_copy, logits2, pred2, partials = _pass1(
        h_flat, Wr.reshape(D, 1), W1.astype(jnp.bfloat16),
        b1.reshape(1, DQ), W2.reshape(DQ, 1),
        jnp.asarray(b2, jnp.float32).reshape(1, 1))

    flat_idx, gate, psel = _select(logits2.reshape(N), pred2.reshape(N))

    sel = _gather(h_flat, flat_idx)
    upd = _mlp(sel, gate.reshape(NSEL, 1), norm_w.reshape(1, D),
               Wg.astype(jnp.bfloat16), Wu.astype(jnp.bfloat16),
               Wd.astype(jnp.bfloat16))

    states_ref = jax.new_ref(states_copy)
    _scatter(upd, flat_idx, states_ref)
    new_states = states_ref[...].reshape(B, T, D)

    inv_n = jnp.float32(1.0 / N)
    router_bce_loss = (jnp.sum(partials[0]) - jnp.sum(gate)) * inv_n
    predictor_loss = (jnp.sum(partials[1]) - jnp.sum(psel)) * inv_n
    return new_states, router_bce_loss, predictor_loss


# speedup vs baseline: 3.4230x; 3.4230x over previous
import functools

import jax
import jax.numpy as jnp
from jax import lax
from jax.experimental import pallas as pl
from jax.experimental.pallas import tpu as pltpu
from jax.experimental.pallas import tpu_sc as plsc

B, T, D, DFF = 4, 8192, 768, 2048
K = 1024
N = B * T
NSEL = B * K
DQ = D // 4

RB1 = 512
NBLK1 = N // RB1


def _pass1_body(h_ref, wr_ref, w1_ref, b1_ref, w2_ref, b2_ref,
                out_ref, logit_ref, pred_ref, part_ref):
    h = h_ref[...]
    out_ref[...] = h
    logits = jnp.dot(h, wr_ref[...],
                     preferred_element_type=jnp.float32)
    logit_ref[...] = logits
    pre = jnp.dot(h.astype(jnp.bfloat16), w1_ref[...],
                  preferred_element_type=jnp.float32) + b1_ref[...]
    pre = pre * 0.5 * (1.0 + lax.erf(pre * 0.7071067811865476))
    pred = jnp.dot(pre, w2_ref[...],
                   preferred_element_type=jnp.float32) + b2_ref[...]
    pred_ref[...] = pred

    def softplus_lanes(x):
        sp = jnp.maximum(x, 0.0) + jnp.log1p(jnp.exp(-jnp.abs(x)))
        return jnp.sum(sp.reshape(RB1 // 128, 128), axis=0, keepdims=True)

    blk = jnp.concatenate([softplus_lanes(logits), softplus_lanes(pred)], 0)

    @pl.when(pl.program_id(0) == 0)
    def _():
        part_ref[...] = jnp.zeros_like(part_ref)
    part_ref[...] += blk


def _pass1(h_flat, wr2, w1b, b12, w22, b22):
    return pl.pallas_call(
        _pass1_body,
        grid=(NBLK1,),
        in_specs=[
            pl.BlockSpec((RB1, D), lambda i: (i, 0)),
            pl.BlockSpec((D, 1), lambda i: (0, 0)),
            pl.BlockSpec((D, DQ), lambda i: (0, 0)),
            pl.BlockSpec((1, DQ), lambda i: (0, 0)),
            pl.BlockSpec((DQ, 1), lambda i: (0, 0)),
            pl.BlockSpec((1, 1), lambda i: (0, 0)),
        ],
        out_specs=[
            pl.BlockSpec((RB1, D), lambda i: (i, 0)),
            pl.BlockSpec((RB1, 1), lambda i: (i, 0)),
            pl.BlockSpec((RB1, 1), lambda i: (i, 0)),
            pl.BlockSpec((2, 128), lambda i: (0, 0)),
        ],
        out_shape=[
            jax.ShapeDtypeStruct((N, D), jnp.float32),
            jax.ShapeDtypeStruct((N, 1), jnp.float32),
            jax.ShapeDtypeStruct((N, 1), jnp.float32),
            jax.ShapeDtypeStruct((2, 128), jnp.float32),
        ],
    )(h_flat, wr2, w1b, b12, w22, b22)


@functools.lru_cache(maxsize=None)
def _sc_mesh():
    return plsc.VectorSubcoreMesh(core_axis_name="c", subcore_axis_name="s",
                                  num_cores=2, num_subcores=16)


_INT_MIN = -2**31


def _f32_key(v):
    b = plsc.bitcast(v, jnp.int32)
    return b ^ ((b >> 31) & jnp.int32(0x7FFFFFFF))


def _select_body(logits_hbm, pred_hbm, idx_out, gate_out, psel_out,
                 lrow, prow, idxs, vals, pvals):
    wid = lax.axis_index("s") * 2 + lax.axis_index("c")

    @pl.when(wid < B)
    def _():
        row = wid
        pltpu.sync_copy(logits_hbm.at[pl.ds(row * T, T)], lrow)
        pltpu.sync_copy(pred_hbm.at[pl.ds(row * T, T)], prow)

        nv = T // 16

        def count_ge(thresh):
            tv = jnp.full((16,), thresh, jnp.int32)

            def body(i, acc):
                key = _f32_key(lrow[pl.ds(i * 16, 16)])
                return acc + jnp.where(key >= tv, 1, 0).astype(jnp.int32)

            acc = lax.fori_loop(0, nv, body,
                                jnp.zeros((16,), jnp.int32), unroll=8)
            return jnp.sum(acc)

        prefix = jnp.where(count_ge(jnp.int32(0)) >= K,
                           jnp.int32(0), jnp.int32(_INT_MIN))

        def bit_body(j, prefix):
            test = prefix + (jnp.int32(1) << (jnp.int32(30) - j))
            return jnp.where(count_ge(test) >= K, test, prefix)

        tau = lax.fori_loop(0, 31, bit_body, prefix)
        c1 = count_ge(tau + 1)
        need = K - c1

        tau_v = jnp.full((16,), tau, jnp.int32)
        lane = lax.broadcasted_iota(jnp.int32, (16,), 0)

        def sweep(i, carry):
            s, eqc = carry
            v = lrow[pl.ds(i * 16, 16)]
            p = prow[pl.ds(i * 16, 16)]
            key = _f32_key(v)
            gt = key > tau_v
            eq = key == tau_v
            eqi = jnp.where(eq, 1, 0).astype(jnp.int32)
            eqpref = plsc.cumsum(eqi)
            sel = gt | (eq & ((eqc + eqpref) <= need))
            seli = jnp.where(sel, 1, 0).astype(jnp.int32)
            spref = plsc.cumsum(seli)
            pos = s + spref - 1
            tok = lane + (i * 16 + row * T)
            plsc.store_scatter(vals, [pos], v, mask=sel)
            plsc.store_scatter(pvals, [pos], p, mask=sel)
            plsc.store_scatter(idxs, [pos], tok, mask=sel)
            return s + jnp.sum(seli), eqc + jnp.sum(eqi)

        lax.fori_loop(0, nv, sweep, (jnp.int32(0), jnp.int32(0)), unroll=4)

        pltpu.sync_copy(idxs, idx_out.at[pl.ds(row * K, K)])
        pltpu.sync_copy(vals, gate_out.at[pl.ds(row * K, K)])
        pltpu.sync_copy(pvals, psel_out.at[pl.ds(row * K, K)])


@functools.lru_cache(maxsize=None)
def _select():
    return pl.kernel(
        _select_body,
        out_type=(
            jax.ShapeDtypeStruct((NSEL,), jnp.int32),
            jax.ShapeDtypeStruct((NSEL,), jnp.float32),
            jax.ShapeDtypeStruct((NSEL,), jnp.float32),
        ),
        mesh=_sc_mesh(),
        scratch_types=[
            pltpu.VMEM((T,), jnp.float32),
            pltpu.VMEM((T,), jnp.float32),
            pltpu.VMEM((K,), jnp.int32),
            pltpu.VMEM((K,), jnp.float32),
            pltpu.VMEM((K,), jnp.float32),
        ],
        compiler_params=pltpu.CompilerParams(needs_layout_passes=False),
    )


RPW = NSEL // 32


def _gather_body(hid_hbm, idx_hbm, out_hbm, idx_v, rows_v, sem):
    wid = lax.axis_index("s") * 2 + lax.axis_index("c")
    base = wid * RPW
    pltpu.sync_copy(idx_hbm.at[pl.ds(base, RPW)], idx_v)
    pltpu.async_copy(hid_hbm.at[idx_v], rows_v, sem).wait()
    pltpu.sync_copy(rows_v, out_hbm.at[pl.ds(base, RPW)])


@functools.lru_cache(maxsize=None)
def _gather():
    return pl.kernel(
        _gather_body,
        out_type=jax.ShapeDtypeStruct((NSEL, D), jnp.float32),
        mesh=_sc_mesh(),
        scratch_types=[
            pltpu.VMEM((RPW,), jnp.int32),
            pltpu.VMEM((RPW, D), jnp.float32),
            pltpu.SemaphoreType.DMA,
        ],
    )


def _scatter_body(upd_hbm, idx_hbm, states_ref, idx_v, rows_v, sem):
    wid = lax.axis_index("s") * 2 + lax.axis_index("c")
    base = wid * RPW
    pltpu.sync_copy(idx_hbm.at[pl.ds(base, RPW)], idx_v)
    pltpu.sync_copy(upd_hbm.at[pl.ds(base, RPW)], rows_v)
    pltpu.async_copy(rows_v, states_ref.at[idx_v], sem).wait()


@functools.lru_cache(maxsize=None)
def _scatter():
    return pl.kernel(
        _scatter_body,
        out_type=(),
        mesh=_sc_mesh(),
        scratch_types=[
            pltpu.VMEM((RPW,), jnp.int32),
            pltpu.VMEM((RPW, D), jnp.float32),
            pltpu.SemaphoreType.DMA,
        ],
    )


RB2 = 512
NBLK2 = NSEL // RB2


def _mlp_body(x_ref, g_ref, nw_ref, wg_ref, wu_ref, wd_ref, out_ref):
    x = x_ref[...]
    var = jnp.mean(x * x, axis=1, keepdims=True)
    h = (x * lax.rsqrt(var + 1e-6) * nw_ref[...]).astype(jnp.bfloat16)
    gproj = jnp.dot(h, wg_ref[...], preferred_element_type=jnp.float32)
    up = jnp.dot(h, wu_ref[...], preferred_element_type=jnp.float32)
    act = (gproj * jax.nn.sigmoid(gproj)) * up
    mlp = jnp.dot(act.astype(jnp.bfloat16), wd_ref[...],
                  preferred_element_type=jnp.float32)
    g = jax.nn.sigmoid(g_ref[...])
    out_ref[...] = x + g * mlp


def _mlp(sel, gate2, nw2, wgb, wub, wdb):
    return pl.pallas_call(
        _mlp_body,
        grid=(NBLK2,),
        in_specs=[
            pl.BlockSpec((RB2, D), lambda i: (i, 0)),
            pl.BlockSpec((RB2, 1), lambda i: (i, 0)),
            pl.BlockSpec((1, D), lambda i: (0, 0)),
            pl.BlockSpec((D, DFF), lambda i: (0, 0)),
            pl.BlockSpec((D, DFF), lambda i: (0, 0)),
            pl.BlockSpec((DFF, D), lambda i: (0, 0)),
        ],
        out_specs=pl.BlockSpec((RB2, D), lambda i: (i, 0)),
        out_shape=jax.ShapeDtypeStruct((NSEL, D), jnp.float32),
    )(sel, gate2, nw2, wgb, wub, wdb)


def kernel(hidden_states, Wr, W1, b1, W2, b2, norm_w, Wg, Wu, Wd, training):
    h_flat = hidden_states.reshape(N, D)
    states_copy, logits2, pred2, partials = _pass1(
        h_flat, Wr.reshape(D, 1), W1.astype(jnp.bfloat16),
        b1.reshape(1, DQ), W2.reshape(DQ, 1),
        jnp.asarray(b2, jnp.float32).reshape(1, 1))

    flat_idx, gate, psel = _select()(logits2.reshape(N), pred2.reshape(N))

    sel = _gather()(h_flat, flat_idx)
    upd = _mlp(sel, gate.reshape(NSEL, 1), norm_w.reshape(1, D),
               Wg.astype(jnp.bfloat16), Wu.astype(jnp.bfloat16),
               Wd.astype(jnp.bfloat16))

    states_ref = jax.new_ref(states_copy)
    _scatter()(upd, flat_idx, states_ref)
    new_states = states_ref[...].reshape(B, T, D)

    inv_n = jnp.float32(1.0 / N)
    router_bce_loss = (jnp.sum(partials[0]) - jnp.sum(gate)) * inv_n
    predictor_loss = (jnp.sum(partials[1]) - jnp.sum(psel)) * inv_n
    return new_states, router_bce_loss, predictor_loss


# speedup vs baseline: 3.6266x; 1.0595x over previous
import functools

import jax
import jax.numpy as jnp
from jax import lax
from jax.experimental import pallas as pl
from jax.experimental.pallas import tpu as pltpu
from jax.experimental.pallas import tpu_sc as plsc

B, T, D, DFF = 4, 8192, 768, 2048
K = 1024
N = B * T
NSEL = B * K
DQ = D // 4

RB1 = 512
NBLK1 = N // RB1


def _pass1_body(h_ref, wr_ref, w1_ref, b1_ref, w2_ref, b2_ref,
                out_ref, logit_ref, pred_ref, part_ref):
    h = h_ref[...]
    out_ref[...] = h
    logits = jnp.dot(h, wr_ref[...],
                     preferred_element_type=jnp.float32)
    logit_ref[...] = logits
    pre = jnp.dot(h.astype(jnp.bfloat16), w1_ref[...],
                  preferred_element_type=jnp.float32) + b1_ref[...]
    pre = pre * 0.5 * (1.0 + lax.erf(pre * 0.7071067811865476))
    pred = jnp.dot(pre, w2_ref[...],
                   preferred_element_type=jnp.float32) + b2_ref[...]
    pred_ref[...] = pred

    def softplus_lanes(x):
        xl = x.reshape(RB1 // 128, 128)
        sp = jnp.maximum(xl, 0.0) + jnp.log1p(jnp.exp(-jnp.abs(xl)))
        return jnp.sum(sp, axis=0, keepdims=True)

    blk = jnp.concatenate([softplus_lanes(logits), softplus_lanes(pred)], 0)

    @pl.when(pl.program_id(0) == 0)
    def _():
        part_ref[...] = jnp.zeros_like(part_ref)
    part_ref[...] += blk


def _pass1(h_flat, wr2, w1b, b12, w22, b22):
    return pl.pallas_call(
        _pass1_body,
        grid=(NBLK1,),
        in_specs=[
            pl.BlockSpec((RB1, D), lambda i: (i, 0)),
            pl.BlockSpec((D, 1), lambda i: (0, 0)),
            pl.BlockSpec((D, DQ), lambda i: (0, 0)),
            pl.BlockSpec((1, DQ), lambda i: (0, 0)),
            pl.BlockSpec((DQ, 1), lambda i: (0, 0)),
            pl.BlockSpec((1, 1), lambda i: (0, 0)),
        ],
        out_specs=[
            pl.BlockSpec((RB1, D), lambda i: (i, 0)),
            pl.BlockSpec((RB1, 1), lambda i: (i, 0)),
            pl.BlockSpec((RB1, 1), lambda i: (i, 0)),
            pl.BlockSpec((2, 128), lambda i: (0, 0)),
        ],
        out_shape=[
            jax.ShapeDtypeStruct((N, D), jnp.float32),
            jax.ShapeDtypeStruct((N, 1), jnp.float32),
            jax.ShapeDtypeStruct((N, 1), jnp.float32),
            jax.ShapeDtypeStruct((2, 128), jnp.float32),
        ],
    )(h_flat, wr2, w1b, b12, w22, b22)


@functools.lru_cache(maxsize=None)
def _sc_mesh():
    return plsc.VectorSubcoreMesh(core_axis_name="c", subcore_axis_name="s",
                                  num_cores=2, num_subcores=16)


_INT_MIN = -2**31


def _f32_key(v):
    b = plsc.bitcast(v, jnp.int32)
    return b ^ ((b >> 31) & jnp.int32(0x7FFFFFFF))


RPW = NSEL // 32


def _selgat_body(logits_hbm, pred_hbm, hid_hbm,
                 idx_out, gate_out, psel_out, sel_out,
                 lrow, prow, kbuf, idxs, vals, pvals, idx_sh,
                 idx_v, rows_v, sem):
    cid = lax.axis_index("c")
    sid = lax.axis_index("s")
    nv = T // 16

    @pl.when((sid == 0) | (sid == 8))
    def _():
        g = sid // 8
        row = cid + 2 * g
        pltpu.sync_copy(logits_hbm.at[pl.ds(row * T, T)], lrow)
        pltpu.sync_copy(pred_hbm.at[pl.ds(row * T, T)], prow)

        def keys_body(i, _):
            kbuf[pl.ds(i * 16, 16)] = _f32_key(lrow[pl.ds(i * 16, 16)])
            return 0

        lax.fori_loop(0, nv, keys_body, 0, unroll=8)

        def count_ge(thresh):
            tv = jnp.full((16,), thresh, jnp.int32)

            def body(i, acc):
                key = kbuf[pl.ds(i * 16, 16)]
                return acc + jnp.where(key >= tv, 1, 0).astype(jnp.int32)

            acc = lax.fori_loop(0, nv, body,
                                jnp.zeros((16,), jnp.int32), unroll=8)
            return jnp.sum(acc)

        prefix = jnp.where(count_ge(jnp.int32(0)) >= K,
                           jnp.int32(0), jnp.int32(_INT_MIN))

        def bit_body(j, prefix):
            test = prefix + (jnp.int32(1) << (jnp.int32(30) - j))
            return jnp.where(count_ge(test) >= K, test, prefix)

        tau = lax.fori_loop(0, 31, bit_body, prefix)
        c1 = count_ge(tau + 1)
        need = K - c1

        tau_v = jnp.full((16,), tau, jnp.int32)
        lane = lax.broadcasted_iota(jnp.int32, (16,), 0)

        def sweep(i, carry):
            s, eqc = carry
            v = lrow[pl.ds(i * 16, 16)]
            p = prow[pl.ds(i * 16, 16)]
            key = kbuf[pl.ds(i * 16, 16)]
            gt = key > tau_v
            eq = key == tau_v
            eqi = jnp.where(eq, 1, 0).astype(jnp.int32)
            eqpref = plsc.cumsum(eqi)
            sel = gt | (eq & ((eqc + eqpref) <= need))
            seli = jnp.where(sel, 1, 0).astype(jnp.int32)
            spref = plsc.cumsum(seli)
            pos = s + spref - 1
            tok = lane + (i * 16 + row * T)
            plsc.store_scatter(vals, [pos], v, mask=sel)
            plsc.store_scatter(pvals, [pos], p, mask=sel)
            plsc.store_scatter(idxs, [pos], tok, mask=sel)
            return s + jnp.sum(seli), eqc + jnp.sum(eqi)

        lax.fori_loop(0, nv, sweep, (jnp.int32(0), jnp.int32(0)), unroll=4)

        pltpu.sync_copy(idxs, idx_sh.at[pl.ds(g * K, K)])
        pltpu.sync_copy(idxs, idx_out.at[pl.ds(row * K, K)])
        pltpu.sync_copy(vals, gate_out.at[pl.ds(row * K, K)])
        pltpu.sync_copy(pvals, psel_out.at[pl.ds(row * K, K)])

    plsc.subcore_barrier()
    gl = sid // 8
    within = (sid % 8) * RPW
    gslot = (cid + 2 * gl) * K + within
    pltpu.sync_copy(idx_sh.at[pl.ds(sid * RPW, RPW)], idx_v)
    pltpu.async_copy(hid_hbm.at[idx_v], rows_v, sem).wait()
    pltpu.sync_copy(rows_v, sel_out.at[pl.ds(gslot, RPW)])


@functools.lru_cache(maxsize=None)
def _selgat():
    return pl.kernel(
        _selgat_body,
        out_type=(
            jax.ShapeDtypeStruct((NSEL,), jnp.int32),
            jax.ShapeDtypeStruct((NSEL,), jnp.float32),
            jax.ShapeDtypeStruct((NSEL,), jnp.float32),
            jax.ShapeDtypeStruct((NSEL, D), jnp.float32),
        ),
        mesh=_sc_mesh(),
        scratch_types=[
            pltpu.VMEM((T,), jnp.float32),
            pltpu.VMEM((T,), jnp.float32),
            pltpu.VMEM((T,), jnp.int32),
            pltpu.VMEM((K,), jnp.int32),
            pltpu.VMEM((K,), jnp.float32),
            pltpu.VMEM((K,), jnp.float32),
            pltpu.VMEM_SHARED((2 * K,), jnp.int32),
            pltpu.VMEM((RPW,), jnp.int32),
            pltpu.VMEM((RPW, D), jnp.float32),
            pltpu.SemaphoreType.DMA,
        ],
        compiler_params=pltpu.CompilerParams(needs_layout_passes=False),
    )


def _scatter_body(upd_hbm, idx_hbm, states_ref, idx_v, rows_v, sem):
    wid = lax.axis_index("s") * 2 + lax.axis_index("c")
    base = wid * RPW
    pltpu.sync_copy(idx_hbm.at[pl.ds(base, RPW)], idx_v)
    pltpu.sync_copy(upd_hbm.at[pl.ds(base, RPW)], rows_v)
    pltpu.async_copy(rows_v, states_ref.at[idx_v], sem).wait()


@functools.lru_cache(maxsize=None)
def _scatter():
    return pl.kernel(
        _scatter_body,
        out_type=(),
        mesh=_sc_mesh(),
        scratch_types=[
            pltpu.VMEM((RPW,), jnp.int32),
            pltpu.VMEM((RPW, D), jnp.float32),
            pltpu.SemaphoreType.DMA,
        ],
    )


RB2 = 512
NBLK2 = NSEL // RB2


def _mlp_body(x_ref, g_ref, nw_ref, wg_ref, wu_ref, wd_ref, out_ref):
    x = x_ref[...]
    var = jnp.mean(x * x, axis=1, keepdims=True)
    h = (x * lax.rsqrt(var + 1e-6) * nw_ref[...]).astype(jnp.bfloat16)
    gproj = jnp.dot(h, wg_ref[...], preferred_element_type=jnp.float32)
    up = jnp.dot(h, wu_ref[...], preferred_element_type=jnp.float32)
    act = (gproj * jax.nn.sigmoid(gproj)) * up
    mlp = jnp.dot(act.astype(jnp.bfloat16), wd_ref[...],
                  preferred_element_type=jnp.float32)
    g = jax.nn.sigmoid(g_ref[...])
    out_ref[...] = x + g * mlp


def _mlp(sel, gate2, nw2, wgb, wub, wdb):
    return pl.pallas_call(
        _mlp_body,
        grid=(NBLK2,),
        in_specs=[
            pl.BlockSpec((RB2, D), lambda i: (i, 0)),
            pl.BlockSpec((RB2, 1), lambda i: (i, 0)),
            pl.BlockSpec((1, D), lambda i: (0, 0)),
            pl.BlockSpec((D, DFF), lambda i: (0, 0)),
            pl.BlockSpec((D, DFF), lambda i: (0, 0)),
            pl.BlockSpec((DFF, D), lambda i: (0, 0)),
        ],
        out_specs=pl.BlockSpec((RB2, D), lambda i: (i, 0)),
        out_shape=jax.ShapeDtypeStruct((NSEL, D), jnp.float32),
    )(sel, gate2, nw2, wgb, wub, wdb)


def kernel(hidden_states, Wr, W1, b1, W2, b2, norm_w, Wg, Wu, Wd, training):
    h_flat = hidden_states.reshape(N, D)
    states_copy, logits2, pred2, partials = _pass1(
        h_flat, Wr.reshape(D, 1), W1.astype(jnp.bfloat16),
        b1.reshape(1, DQ), W2.reshape(DQ, 1),
        jnp.asarray(b2, jnp.float32).reshape(1, 1))

    flat_idx, gate, psel, sel = _selgat()(
        logits2.reshape(N), pred2.reshape(N), h_flat)
    upd = _mlp(sel, gate.reshape(NSEL, 1), norm_w.reshape(1, D),
               Wg.astype(jnp.bfloat16), Wu.astype(jnp.bfloat16),
               Wd.astype(jnp.bfloat16))

    states_ref = jax.new_ref(states_copy)
    _scatter()(upd, flat_idx, states_ref)
    new_states = states_ref[...].reshape(B, T, D)

    inv_n = jnp.float32(1.0 / N)
    router_bce_loss = (jnp.sum(partials[0]) - jnp.sum(gate)) * inv_n
    predictor_loss = (jnp.sum(partials[1]) - jnp.sum(psel)) * inv_n
    return new_states, router_bce_loss, predictor_loss


# speedup vs baseline: 4.6921x; 1.2938x over previous
import functools

import jax
import jax.numpy as jnp
from jax import lax
from jax.experimental import pallas as pl
from jax.experimental.pallas import tpu as pltpu
from jax.experimental.pallas import tpu_sc as plsc

B, T, D, DFF = 4, 8192, 768, 2048
K = 1024
N = B * T
NSEL = B * K
DQ = D // 4
NH = N // 2
KH = 2 * K

RB1 = 1024
NBLK1 = NH // RB1


def _pass1_body(h_ref, wr_ref, w1_ref, b1_ref, w2_ref, b2_ref,
                out_ref, logit_ref, pred_ref, part_ref):
    h = h_ref[...]
    out_ref[...] = h
    logits = jnp.dot(h, wr_ref[...],
                     preferred_element_type=jnp.float32)
    logit_ref[...] = logits.reshape(RB1 // 128, 128)
    pre = jnp.dot(h.astype(jnp.bfloat16), w1_ref[...],
                  preferred_element_type=jnp.float32) + b1_ref[...]
    pre = pre * 0.5 * (1.0 + lax.erf(pre * 0.7071067811865476))
    pred = jnp.dot(pre, w2_ref[...],
                   preferred_element_type=jnp.float32) + b2_ref[...]
    pred_ref[...] = pred.reshape(RB1 // 128, 128)

    def softplus_lanes(xl):
        sp = jnp.maximum(xl, 0.0) + jnp.log1p(jnp.exp(-jnp.abs(xl)))
        return jnp.sum(sp, axis=0, keepdims=True)

    blk = jnp.concatenate([softplus_lanes(logit_ref[...]),
                           softplus_lanes(pred_ref[...])], 0)

    @pl.when(pl.program_id(0) == 0)
    def _():
        part_ref[...] = jnp.zeros_like(part_ref)
    part_ref[...] += blk


@functools.lru_cache(maxsize=None)
def _pass1(half):
    off = half * NBLK1
    body = _pass1_body
    if half:
        def body(prev_ref, *refs):
            del prev_ref
            _pass1_body(*refs)

    prev_spec = [pl.BlockSpec(memory_space=pl.ANY)] if half else []

    def call(*args):
        return pl.pallas_call(
            body,
            grid=(NBLK1,),
            in_specs=prev_spec + [
                pl.BlockSpec((RB1, D), lambda i: (i + off, 0)),
                pl.BlockSpec((D, 1), lambda i: (0, 0)),
                pl.BlockSpec((D, DQ), lambda i: (0, 0)),
                pl.BlockSpec((1, DQ), lambda i: (0, 0)),
                pl.BlockSpec((DQ, 1), lambda i: (0, 0)),
                pl.BlockSpec((1, 1), lambda i: (0, 0)),
            ],
            out_specs=[
                pl.BlockSpec((RB1, D), lambda i: (i + off, 0)),
                pl.BlockSpec((RB1 // 128, 128), lambda i: (i, 0)),
                pl.BlockSpec((RB1 // 128, 128), lambda i: (i, 0)),
                pl.BlockSpec((2, 128), lambda i: (0, 0)),
            ],
            out_shape=[
                jax.ShapeDtypeStruct((N, D), jnp.float32),
                jax.ShapeDtypeStruct((NH // 128, 128), jnp.float32),
                jax.ShapeDtypeStruct((NH // 128, 128), jnp.float32),
                jax.ShapeDtypeStruct((2, 128), jnp.float32),
            ],
            input_output_aliases={0: 0} if half else {},
        )(*args)

    return call


@functools.lru_cache(maxsize=None)
def _sc_mesh():
    return plsc.VectorSubcoreMesh(core_axis_name="c", subcore_axis_name="s",
                                  num_cores=2, num_subcores=16)


_INT_MIN = -2**31
RPW = KH // 32


def _f32_key(v):
    b = plsc.bitcast(v, jnp.int32)
    return b ^ ((b >> 31) & jnp.int32(0x7FFFFFFF))


def _selgat_body(pair, logits_hbm, pred_hbm, hid_hbm,
                 idx_out, gate_out, psel_out, sel_out,
                 lrow, prow, kbuf, idxs, vals, pvals, idx_sh,
                 idx_v, rows_v, sem):
    cid = lax.axis_index("c")
    sid = lax.axis_index("s")
    nv = T // 16

    @pl.when(sid == 0)
    def _():
        row = cid
        pltpu.sync_copy(logits_hbm.at[pl.ds(row * T, T)], lrow)
        pltpu.sync_copy(pred_hbm.at[pl.ds(row * T, T)], prow)

        def keys_body(i, _):
            kbuf[pl.ds(i * 16, 16)] = _f32_key(lrow[pl.ds(i * 16, 16)])
            return 0

        lax.fori_loop(0, nv, keys_body, 0, unroll=8)

        def count_ge(thresh):
            tv = jnp.full((16,), thresh, jnp.int32)

            def body(i, acc):
                key = kbuf[pl.ds(i * 16, 16)]
                return acc + jnp.where(key >= tv, 1, 0).astype(jnp.int32)

            acc = lax.fori_loop(0, nv, body,
                                jnp.zeros((16,), jnp.int32), unroll=8)
            return jnp.sum(acc)

        prefix = jnp.where(count_ge(jnp.int32(0)) >= K,
                           jnp.int32(0), jnp.int32(_INT_MIN))

        def bit_body(j, prefix):
            test = prefix + (jnp.int32(1) << (jnp.int32(30) - j))
            return jnp.where(count_ge(test) >= K, test, prefix)

        tau = lax.fori_loop(0, 31, bit_body, prefix)
        c1 = count_ge(tau + 1)
        need = K - c1

        tau_v = jnp.full((16,), tau, jnp.int32)
        lane = lax.broadcasted_iota(jnp.int32, (16,), 0)
        tok_base = (pair * 2 + row) * T

        def sweep(i, carry):
            s, eqc = carry
            v = lrow[pl.ds(i * 16, 16)]
            p = prow[pl.ds(i * 16, 16)]
            key = kbuf[pl.ds(i * 16, 16)]
            gt = key > tau_v
            eq = key == tau_v
            eqi = jnp.where(eq, 1, 0).astype(jnp.int32)
            eqpref = plsc.cumsum(eqi)
            sel = gt | (eq & ((eqc + eqpref) <= need))
            seli = jnp.where(sel, 1, 0).astype(jnp.int32)
            spref = plsc.cumsum(seli)
            pos = s + spref - 1
            tok = lane + (i * 16 + tok_base)
            plsc.store_scatter(vals, [pos], v, mask=sel)
            plsc.store_scatter(pvals, [pos], p, mask=sel)
            plsc.store_scatter(idxs, [pos], tok, mask=sel)
            return s + jnp.sum(seli), eqc + jnp.sum(eqi)

        lax.fori_loop(0, nv, sweep, (jnp.int32(0), jnp.int32(0)), unroll=4)

        pltpu.sync_copy(idxs, idx_sh)
        pltpu.sync_copy(idxs, idx_out.at[pl.ds(row * K, K)])
        pltpu.sync_copy(vals, gate_out.at[pl.ds(row * K, K)])
        pltpu.sync_copy(pvals, psel_out.at[pl.ds(row * K, K)])

    plsc.subcore_barrier()
    gslot = cid * K + sid * RPW
    pltpu.sync_copy(idx_sh.at[pl.ds(sid * RPW, RPW)], idx_v)
    pltpu.async_copy(hid_hbm.at[idx_v], rows_v, sem).wait()
    pltpu.sync_copy(rows_v, sel_out.at[pl.ds(gslot, RPW)])


@functools.lru_cache(maxsize=None)
def _selgat(pair):
    return pl.kernel(
        functools.partial(_selgat_body, pair),
        out_type=(
            jax.ShapeDtypeStruct((KH,), jnp.int32),
            jax.ShapeDtypeStruct((KH,), jnp.float32),
            jax.ShapeDtypeStruct((KH,), jnp.float32),
            jax.ShapeDtypeStruct((KH, D), jnp.float32),
        ),
        mesh=_sc_mesh(),
        scratch_types=[
            pltpu.VMEM((T,), jnp.float32),
            pltpu.VMEM((T,), jnp.float32),
            pltpu.VMEM((T,), jnp.int32),
            pltpu.VMEM((K,), jnp.int32),
            pltpu.VMEM((K,), jnp.float32),
            pltpu.VMEM((K,), jnp.float32),
            pltpu.VMEM_SHARED((K,), jnp.int32),
            pltpu.VMEM((RPW,), jnp.int32),
            pltpu.VMEM((RPW, D), jnp.float32),
            pltpu.SemaphoreType.DMA,
        ],
        compiler_params=pltpu.CompilerParams(needs_layout_passes=False),
    )


def _scatter_body(upd_hbm, idx_hbm, states_ref, idx_v, rows_v, sem):
    wid = lax.axis_index("s") * 2 + lax.axis_index("c")
    base = wid * RPW
    pltpu.sync_copy(idx_hbm.at[pl.ds(base, RPW)], idx_v)
    pltpu.sync_copy(upd_hbm.at[pl.ds(base, RPW)], rows_v)
    pltpu.async_copy(rows_v, states_ref.at[idx_v], sem).wait()


@functools.lru_cache(maxsize=None)
def _scatter():
    return pl.kernel(
        _scatter_body,
        out_type=(),
        mesh=_sc_mesh(),
        scratch_types=[
            pltpu.VMEM((RPW,), jnp.int32),
            pltpu.VMEM((RPW, D), jnp.float32),
            pltpu.SemaphoreType.DMA,
        ],
    )


RB2 = 1024
NBLK2 = KH // RB2


def _mlp_body(x_ref, g_ref, nw_ref, wg_ref, wu_ref, wd_ref, out_ref):
    x = x_ref[...]
    var = jnp.mean(x * x, axis=1, keepdims=True)
    h = (x * lax.rsqrt(var + 1e-6) * nw_ref[...]).astype(jnp.bfloat16)
    gproj = jnp.dot(h, wg_ref[...], preferred_element_type=jnp.float32)
    up = jnp.dot(h, wu_ref[...], preferred_element_type=jnp.float32)
    act = (gproj * jax.nn.sigmoid(gproj)) * up
    mlp = jnp.dot(act.astype(jnp.bfloat16), wd_ref[...],
                  preferred_element_type=jnp.float32)
    g = jax.nn.sigmoid(g_ref[...])
    out_ref[...] = x + g * mlp


def _mlp(sel, gate2, nw2, wgb, wub, wdb):
    return pl.pallas_call(
        _mlp_body,
        grid=(NBLK2,),
        in_specs=[
            pl.BlockSpec((RB2, D), lambda i: (i, 0)),
            pl.BlockSpec((RB2, 1), lambda i: (i, 0)),
            pl.BlockSpec((1, D), lambda i: (0, 0)),
            pl.BlockSpec((D, DFF), lambda i: (0, 0)),
            pl.BlockSpec((D, DFF), lambda i: (0, 0)),
            pl.BlockSpec((DFF, D), lambda i: (0, 0)),
        ],
        out_specs=pl.BlockSpec((RB2, D), lambda i: (i, 0)),
        out_shape=jax.ShapeDtypeStruct((KH, D), jnp.float32),
    )(sel, gate2, nw2, wgb, wub, wdb)


def kernel(hidden_states, Wr, W1, b1, W2, b2, norm_w, Wg, Wu, Wd, training):
    h_flat = hidden_states.reshape(N, D)
    wr2 = Wr.reshape(D, 1)
    w1b = W1.astype(jnp.bfloat16)
    b12 = b1.reshape(1, DQ)
    w22 = W2.reshape(DQ, 1)
    b22 = jnp.asarray(b2, jnp.float32).reshape(1, 1)
    nw2 = norm_w.reshape(1, D)
    wgb = Wg.astype(jnp.bfloat16)
    wub = Wu.astype(jnp.bfloat16)
    wdb = Wd.astype(jnp.bfloat16)

    states_a, logits_a, pred_a, part_a = _pass1(0)(
        h_flat, wr2, w1b, b12, w22, b22)
    idx_a, gate_a, psel_a, sel_a = _selgat(0)(
        logits_a.reshape(NH), pred_a.reshape(NH), h_flat)

    states, logits_b, pred_b, part_b = _pass1(1)(
        states_a, h_flat, wr2, w1b, b12, w22, b22)
    idx_b, gate_b, psel_b, sel_b = _selgat(1)(
        logits_b.reshape(NH), pred_b.reshape(NH), h_flat)

    upd_a = _mlp(sel_a, gate_a.reshape(KH, 1), nw2, wgb, wub, wdb)
    upd_b = _mlp(sel_b, gate_b.reshape(KH, 1), nw2, wgb, wub, wdb)

    states_ref = jax.new_ref(states)
    _scatter()(upd_a, idx_a, states_ref)
    _scatter()(upd_b, idx_b, states_ref)
    new_states = states_ref[...].reshape(B, T, D)

    inv_n = jnp.float32(1.0 / N)
    sp = jnp.sum(part_a, axis=1) + jnp.sum(part_b, axis=1)
    router_bce_loss = (sp[0] - jnp.sum(gate_a) - jnp.sum(gate_b)) * inv_n
    predictor_loss = (sp[1] - jnp.sum(psel_a) - jnp.sum(psel_b)) * inv_n
    return new_states, router_bce_loss, predictor_loss


# speedup vs baseline: 4.7113x; 1.0041x over previous
import functools

import jax
import jax.numpy as jnp
from jax import lax
from jax.experimental import pallas as pl
from jax.experimental.pallas import tpu as pltpu
from jax.experimental.pallas import tpu_sc as plsc

B, T, D, DFF = 4, 8192, 768, 2048
K = 1024
N = B * T
NSEL = B * K
DQ = D // 4
NH = N // 2
KH = 2 * K

RB1 = 1024
NBLK1 = NH // RB1


def _pass1_body(h_ref, wr_ref, w1_ref, b1_ref, w2_ref, b2_ref,
                out_ref, logit_ref, pred_ref, part_ref):
    h = h_ref[...]
    out_ref[...] = h
    logits = jnp.dot(h, wr_ref[...],
                     preferred_element_type=jnp.float32)
    logit_ref[...] = logits.reshape(RB1 // 128, 128)
    pre = jnp.dot(h.astype(jnp.bfloat16), w1_ref[...],
                  preferred_element_type=jnp.float32) + b1_ref[...]
    pre = pre * 0.5 * (1.0 + lax.erf(pre * 0.7071067811865476))
    pred = jnp.dot(pre, w2_ref[...],
                   preferred_element_type=jnp.float32) + b2_ref[...]
    pred_ref[...] = pred.reshape(RB1 // 128, 128)

    def softplus_lanes(xl):
        sp = jnp.maximum(xl, 0.0) + jnp.log1p(jnp.exp(-jnp.abs(xl)))
        return jnp.sum(sp, axis=0, keepdims=True)

    blk = jnp.concatenate([softplus_lanes(logit_ref[...]),
                           softplus_lanes(pred_ref[...])], 0)

    @pl.when(pl.program_id(0) == 0)
    def _():
        part_ref[...] = jnp.zeros_like(part_ref)
    part_ref[...] += blk


@functools.lru_cache(maxsize=None)
def _pass1(half):
    off = half * NBLK1
    body = _pass1_body
    if half:
        def body(prev_ref, *refs):
            del prev_ref
            _pass1_body(*refs)

    prev_spec = [pl.BlockSpec(memory_space=pl.ANY)] if half else []

    def call(*args):
        return pl.pallas_call(
            body,
            grid=(NBLK1,),
            in_specs=prev_spec + [
                pl.BlockSpec((RB1, D), lambda i: (i + off, 0)),
                pl.BlockSpec((D, 1), lambda i: (0, 0)),
                pl.BlockSpec((D, DQ), lambda i: (0, 0)),
                pl.BlockSpec((1, DQ), lambda i: (0, 0)),
                pl.BlockSpec((DQ, 1), lambda i: (0, 0)),
                pl.BlockSpec((1, 1), lambda i: (0, 0)),
            ],
            out_specs=[
                pl.BlockSpec((RB1, D), lambda i: (i + off, 0)),
                pl.BlockSpec((RB1 // 128, 128), lambda i: (i, 0)),
                pl.BlockSpec((RB1 // 128, 128), lambda i: (i, 0)),
                pl.BlockSpec((2, 128), lambda i: (0, 0)),
            ],
            out_shape=[
                jax.ShapeDtypeStruct((N, D), jnp.float32),
                jax.ShapeDtypeStruct((NH // 128, 128), jnp.float32),
                jax.ShapeDtypeStruct((NH // 128, 128), jnp.float32),
                jax.ShapeDtypeStruct((2, 128), jnp.float32),
            ],
            input_output_aliases={0: 0} if half else {},
        )(*args)

    return call


@functools.lru_cache(maxsize=None)
def _sc_mesh():
    return plsc.VectorSubcoreMesh(core_axis_name="c", subcore_axis_name="s",
                                  num_cores=2, num_subcores=16)


_INT_MIN = -2**31
RPW = KH // 32


def _f32_key(v):
    b = plsc.bitcast(v, jnp.int32)
    return b ^ ((b >> 31) & jnp.int32(0x7FFFFFFF))


def _selgat_body(pair, logits_hbm, pred_hbm, hid_hbm,
                 idx_out, gate_out, psel_out, sel_out,
                 lrow, prow, kbuf, cand, idxs, vals, pvals, idx_sh,
                 idx_v, rows_v, sem):
    cid = lax.axis_index("c")
    sid = lax.axis_index("s")
    nv = T // 16

    @pl.when(sid == 0)
    def _():
        row = cid
        pltpu.sync_copy(logits_hbm.at[pl.ds(row * T, T)], lrow)
        pltpu.sync_copy(pred_hbm.at[pl.ds(row * T, T)], prow)

        def keys_body(i, _):
            kbuf[pl.ds(i * 16, 16)] = _f32_key(lrow[pl.ds(i * 16, 16)])
            return 0

        lax.fori_loop(0, nv, keys_body, 0, unroll=8)

        def count_ge(thresh):
            tv = jnp.full((16,), thresh, jnp.int32)

            def body(i, acc):
                key = kbuf[pl.ds(i * 16, 16)]
                return acc + jnp.where(key >= tv, 1, 0).astype(jnp.int32)

            acc = lax.fori_loop(0, nv, body,
                                jnp.zeros((16,), jnp.int32), unroll=8)
            return jnp.sum(acc)

        prefix = jnp.where(count_ge(jnp.int32(0)) >= K,
                           jnp.int32(0), jnp.int32(_INT_MIN))

        def bit_hi(j, prefix):
            test = prefix + (jnp.int32(1) << (jnp.int32(30) - j))
            return jnp.where(count_ge(test) >= K, test, prefix)

        prefix = lax.fori_loop(0, 10, bit_hi, prefix)
        hi = prefix + (jnp.int32(1) << 21)
        cnt_hi = count_ge(hi)

        pv = jnp.full((16,), prefix, jnp.int32)
        hv = jnp.full((16,), hi, jnp.int32)
        zero16 = jnp.zeros((16,), jnp.int32)

        def compact(i, s):
            key = kbuf[pl.ds(i * 16, 16)]
            msk = (key >= pv) & (key < hv)
            cpos = s + plsc.cumsum(
                jnp.where(msk, 1, 0).astype(jnp.int32)) - 1
            plsc.store_scatter(cand, [cpos], key, mask=msk)
            return s + plsc.all_reduce_population_count(msk)

        m = jnp.max(lax.fori_loop(0, nv, compact, zero16, unroll=4))
        nfull = m >> 4
        rem_v = jnp.full((16,), m & 15, jnp.int32)
        lane16 = lax.broadcasted_iota(jnp.int32, (16,), 0)

        def count_cand(thresh):
            tv = jnp.full((16,), thresh, jnp.int32)

            def body(i, acc):
                key = cand[pl.ds(i * 16, 16)]
                return acc + jnp.where(key >= tv, 1, 0).astype(jnp.int32)

            acc = lax.fori_loop(0, nfull, body, zero16)
            tail = cand[pl.ds(nfull * 16, 16)]
            acc = acc + jnp.where((lane16 < rem_v) & (tail >= tv),
                                  1, 0).astype(jnp.int32)
            return cnt_hi + jnp.sum(acc)

        def bit_lo(j, prefix):
            test = prefix + (jnp.int32(1) << (jnp.int32(20) - j))
            return jnp.where(count_cand(test) >= K, test, prefix)

        tau = lax.fori_loop(0, 21, bit_lo, prefix)
        c1 = count_cand(tau + 1)
        need = K - c1

        tau_v = jnp.full((16,), tau, jnp.int32)
        need_v = jnp.full((16,), need, jnp.int32)
        lane = lax.broadcasted_iota(jnp.int32, (16,), 0)
        tok_base = (pair * 2 + row) * T
        zero_v = jnp.zeros((16,), jnp.int32)

        def sweep(i, carry):
            s, eqc = carry
            v = lrow[pl.ds(i * 16, 16)]
            p = prow[pl.ds(i * 16, 16)]
            key = kbuf[pl.ds(i * 16, 16)]
            gt = key > tau_v
            eq = key == tau_v
            eqi = jnp.where(eq, 1, 0).astype(jnp.int32)
            eqpref = plsc.cumsum(eqi)
            sel = gt | (eq & ((eqc + eqpref) <= need_v))
            seli = jnp.where(sel, 1, 0).astype(jnp.int32)
            spref = plsc.cumsum(seli)
            pos = s + spref - 1
            tok = lane + (i * 16 + tok_base)
            plsc.store_scatter(vals, [pos], v, mask=sel)
            plsc.store_scatter(pvals, [pos], p, mask=sel)
            plsc.store_scatter(idxs, [pos], tok, mask=sel)
            return (s + plsc.all_reduce_population_count(sel),
                    eqc + plsc.all_reduce_population_count(eq))

        lax.fori_loop(0, nv, sweep, (zero_v, zero_v), unroll=4)

        pltpu.sync_copy(idxs, idx_sh)
        pltpu.sync_copy(idxs, idx_out.at[pl.ds(row * K, K)])
        pltpu.sync_copy(vals, gate_out.at[pl.ds(row * K, K)])
        pltpu.sync_copy(pvals, psel_out.at[pl.ds(row * K, K)])

    plsc.subcore_barrier()
    gslot = cid * K + sid * RPW
    pltpu.sync_copy(idx_sh.at[pl.ds(sid * RPW, RPW)], idx_v)
    pltpu.async_copy(hid_hbm.at[idx_v], rows_v, sem).wait()
    pltpu.sync_copy(rows_v, sel_out.at[pl.ds(gslot, RPW)])


@functools.lru_cache(maxsize=None)
def _selgat(pair):
    return pl.kernel(
        functools.partial(_selgat_body, pair),
        out_type=(
            jax.ShapeDtypeStruct((KH,), jnp.int32),
            jax.ShapeDtypeStruct((KH,), jnp.float32),
            jax.ShapeDtypeStruct((KH,), jnp.float32),
            jax.ShapeDtypeStruct((KH, D), jnp.float32),
        ),
        mesh=_sc_mesh(),
        scratch_types=[
            pltpu.VMEM((T,), jnp.float32),
            pltpu.VMEM((T,), jnp.float32),
            pltpu.VMEM((T,), jnp.int32),
            pltpu.VMEM((T + 16,), jnp.int32),
            pltpu.VMEM((K,), jnp.int32),
            pltpu.VMEM((K,), jnp.float32),
            pltpu.VMEM((K,), jnp.float32),
            pltpu.VMEM_SHARED((K,), jnp.int32),
            pltpu.VMEM((RPW,), jnp.int32),
            pltpu.VMEM((RPW, D), jnp.float32),
            pltpu.SemaphoreType.DMA,
        ],
        compiler_params=pltpu.CompilerParams(needs_layout_passes=False),
    )


def _scatter_body(upd_hbm, idx_hbm, states_ref, idx_v, rows_v, sem):
    wid = lax.axis_index("s") * 2 + lax.axis_index("c")
    base = wid * RPW
    pltpu.sync_copy(idx_hbm.at[pl.ds(base, RPW)], idx_v)
    pltpu.sync_copy(upd_hbm.at[pl.ds(base, RPW)], rows_v)
    pltpu.async_copy(rows_v, states_ref.at[idx_v], sem).wait()


@functools.lru_cache(maxsize=None)
def _scatter():
    return pl.kernel(
        _scatter_body,
        out_type=(),
        mesh=_sc_mesh(),
        scratch_types=[
            pltpu.VMEM((RPW,), jnp.int32),
            pltpu.VMEM((RPW, D), jnp.float32),
            pltpu.SemaphoreType.DMA,
        ],
    )


RB2 = 1024
NBLK2 = KH // RB2


def _mlp_body(x_ref, g_ref, nw_ref, wg_ref, wu_ref, wd_ref, out_ref):
    x = x_ref[...]
    var = jnp.mean(x * x, axis=1, keepdims=True)
    h = (x * lax.rsqrt(var + 1e-6) * nw_ref[...]).astype(jnp.bfloat16)
    gproj = jnp.dot(h, wg_ref[...],
                    preferred_element_type=jnp.float32).astype(jnp.bfloat16)
    up = jnp.dot(h, wu_ref[...],
                 preferred_element_type=jnp.float32).astype(jnp.bfloat16)
    act = (gproj * jax.nn.sigmoid(gproj)) * up
    mlp = jnp.dot(act, wd_ref[...], preferred_element_type=jnp.float32)
    g = jax.nn.sigmoid(g_ref[...])
    out_ref[...] = x + g * mlp


def _mlp(sel, gate2, nw2, wgb, wub, wdb):
    return pl.pallas_call(
        _mlp_body,
        grid=(NBLK2,),
        in_specs=[
            pl.BlockSpec((RB2, D), lambda i: (i, 0)),
            pl.BlockSpec((RB2, 1), lambda i: (i, 0)),
            pl.BlockSpec((1, D), lambda i: (0, 0)),
            pl.BlockSpec((D, DFF), lambda i: (0, 0)),
            pl.BlockSpec((D, DFF), lambda i: (0, 0)),
            pl.BlockSpec((DFF, D), lambda i: (0, 0)),
        ],
        out_specs=pl.BlockSpec((RB2, D), lambda i: (i, 0)),
        out_shape=jax.ShapeDtypeStruct((KH, D), jnp.float32),
    )(sel, gate2, nw2, wgb, wub, wdb)


def kernel(hidden_states, Wr, W1, b1, W2, b2, norm_w, Wg, Wu, Wd, training):
    h_flat = hidden_states.reshape(N, D)
    wr2 = Wr.reshape(D, 1)
    w1b = W1.astype(jnp.bfloat16)
    b12 = b1.reshape(1, DQ)
    w22 = W2.reshape(DQ, 1)
    b22 = jnp.asarray(b2, jnp.float32).reshape(1, 1)
    nw2 = norm_w.reshape(1, D)
    wgb = Wg.astype(jnp.bfloat16)
    wub = Wu.astype(jnp.bfloat16)
    wdb = Wd.astype(jnp.bfloat16)

    states_a, logits_a, pred_a, part_a = _pass1(0)(
        h_flat, wr2, w1b, b12, w22, b22)
    idx_a, gate_a, psel_a, sel_a = _selgat(0)(
        logits_a.reshape(NH), pred_a.reshape(NH), h_flat)

    states, logits_b, pred_b, part_b = _pass1(1)(
        states_a, h_flat, wr2, w1b, b12, w22, b22)
    idx_b, gate_b, psel_b, sel_b = _selgat(1)(
        logits_b.reshape(NH), pred_b.reshape(NH), h_flat)

    upd_a = _mlp(sel_a, gate_a.reshape(KH, 1), nw2, wgb, wub, wdb)
    upd_b = _mlp(sel_b, gate_b.reshape(KH, 1), nw2, wgb, wub, wdb)

    states_ref = jax.new_ref(states)
    _scatter()(upd_a, idx_a, states_ref)
    _scatter()(upd_b, idx_b, states_ref)
    new_states = states_ref[...].reshape(B, T, D)

    inv_n = jnp.float32(1.0 / N)
    sp = jnp.sum(part_a, axis=1) + jnp.sum(part_b, axis=1)
    router_bce_loss = (sp[0] - jnp.sum(gate_a) - jnp.sum(gate_b)) * inv_n
    predictor_loss = (sp[1] - jnp.sum(psel_a) - jnp.sum(psel_b)) * inv_n
    return new_states, router_bce_loss, predictor_loss
